# baseline (speedup 1.0000x reference)
import functools

import jax
import jax.numpy as jnp
from jax.experimental import pallas as pl

B, D, H, E, TOP_K = 2048, 1024, 2048, 8, 2


def _router_body(x_ref, gw_ref, gb_ref, w_ref):
    x = x_ref[...]
    logits = jnp.dot(x, gw_ref[...].T, preferred_element_type=jnp.float32)
    logits = logits + gb_ref[...]
    m = jnp.max(logits, axis=1, keepdims=True)
    ex = jnp.exp(logits - m)
    probs = ex / jnp.sum(ex, axis=1, keepdims=True)
    col = jax.lax.broadcasted_iota(jnp.int32, probs.shape, 1)
    cols = []
    for e in range(E):
        p_e = jnp.sum(jnp.where(col == e, probs, 0.0), axis=1, keepdims=True)
        gt = (probs > p_e).astype(jnp.float32)
        eq_lt = ((probs == p_e) & (col < e)).astype(jnp.float32)
        rank = jnp.sum(gt + eq_lt, axis=1, keepdims=True)
        cols.append(jnp.where(rank < TOP_K, p_e, 0.0))
    w_ref[...] = jnp.concatenate(cols, axis=1)


def _expert_body(xb_ref, w_ref, wv_ref, wg_ref, o_ref):
    e = pl.program_id(1)
    w_all = w_ref[...]
    col = jax.lax.broadcasted_iota(jnp.int32, w_all.shape, 1)
    w = jnp.sum(jnp.where(col == e, w_all, 0.0), axis=1)

    xb = xb_ref[...]
    v = jnp.dot(xb, wv_ref[0], preferred_element_type=jnp.float32)
    g = jnp.dot(xb, wg_ref[0], preferred_element_type=jnp.float32)
    contrib = (v * jax.nn.sigmoid(g)) * w[:, None]

    @pl.when(e == 0)
    def _():
        o_ref[...] = contrib

    @pl.when(e != 0)
    def _():
        o_ref[...] += contrib


@functools.partial(jax.jit, static_argnames=())
def kernel(x, expert_weights_v, expert_weights_g, gate_w, gate_b):
    gb2 = gate_b.reshape(1, E)
    wv_b = expert_weights_v.astype(jnp.bfloat16)
    wg_b = expert_weights_g.astype(jnp.bfloat16)
    xb = x.astype(jnp.bfloat16)

    w_mat = pl.pallas_call(
        _router_body,
        out_shape=jax.ShapeDtypeStruct((B, E), jnp.float32),
    )(x, gate_w, gb2)

    TB = 1024
    NB = B // TB
    return pl.pallas_call(
        _expert_body,
        grid=(NB, E),
        in_specs=[
            pl.BlockSpec((TB, D), lambda nb, e: (nb, 0)),
            pl.BlockSpec((TB, E), lambda nb, e: (nb, 0)),
            pl.BlockSpec((1, D, H), lambda nb, e: (e, 0, 0)),
            pl.BlockSpec((1, D, H), lambda nb, e: (e, 0, 0)),
        ],
        out_specs=pl.BlockSpec((TB, H), lambda nb, e: (nb, 0)),
        out_shape=jax.ShapeDtypeStruct((B, H), jnp.float32),
    )(xb, w_mat, wv_b, wg_b)
